# Initial kernel scaffold; baseline (speedup 1.0000x reference)
import jax
import jax.numpy as jnp
from jax.experimental import pallas as pl


def kernel(x, edge_index, W1, a_src1, a_dst1, W2, a_src2, a_dst2):
    raise NotImplementedError("write your pallas kernel here")


# speedup vs baseline: 7.7102x; 7.7102x over previous
import functools

import jax
import jax.numpy as jnp
from jax import lax
from jax.experimental import pallas as pl
from jax.experimental.pallas import tpu as pltpu
from jax.experimental.pallas import tpu_sc as plsc

_N = 10000
_E = 320000
_DIN = 128
_HID = 128
_NCLS = 16
_H = 8

_NROW_BLK = 1000
_DOFF = 10240
_SDROWS = 20544
_TRASH = 10240
_AROWS = 10304

_STRIP = 2048
_EPAD = 20480
_ETOT = 16 * _EPAD
_NSTRIP = _EPAD // _STRIP
_B = 32
_NBATCH = _STRIP // _B
_BSTASH = 2176


def _proj1_body(x_ref, w_ref, as_ref, ad_ref, zt_ref, s_ref, d_ref):
    z = jnp.dot(x_ref[...], w_ref[...], preferred_element_type=jnp.float32)
    zh = z.reshape(_NROW_BLK, _H, _HID)
    zt_ref[...] = zh.transpose(1, 0, 2)
    es = jnp.sum(zh * as_ref[...][None], axis=-1)
    ed = jnp.sum(zh * ad_ref[...][None], axis=-1)
    s_ref[...] = jnp.repeat(es, 16, axis=1)
    d_ref[...] = jnp.repeat(ed, 16, axis=1)


def _proj1(x, W1, a_src1, a_dst1):
    return pl.pallas_call(
        _proj1_body,
        grid=(_N // _NROW_BLK,),
        in_specs=[
            pl.BlockSpec((_NROW_BLK, _DIN), lambda i: (i, 0)),
            pl.BlockSpec((_DIN, _H * _HID), lambda i: (0, 0)),
            pl.BlockSpec((_H, _HID), lambda i: (0, 0)),
            pl.BlockSpec((_H, _HID), lambda i: (0, 0)),
        ],
        out_specs=[
            pl.BlockSpec((_H, _NROW_BLK, _HID), lambda i: (0, i, 0)),
            pl.BlockSpec((_NROW_BLK, 128), lambda i: (i, 0)),
            pl.BlockSpec((_NROW_BLK, 128), lambda i: (i, 0)),
        ],
        out_shape=[
            jax.ShapeDtypeStruct((_H, _N, _HID), jnp.float32),
            jax.ShapeDtypeStruct((_N, 128), jnp.float32),
            jax.ShapeDtypeStruct((_N, 128), jnp.float32),
        ],
    )(x, W1, a_src1, a_dst1)


def _mid_body(o_ref, sd_ref, w_ref, as_ref, ad_ref, z_ref, s_ref, d_ref):
    s = sd_ref[...].reshape(_NROW_BLK, _H, 16)[:, :, 0]
    r = 1.0 / (s + 1e-16)
    o = o_ref[...]
    hm = jnp.zeros((_NROW_BLK, _HID), jnp.float32)
    for h in range(_H):
        hm = hm + o[h] * r[:, h:h + 1]
    hm = hm * (1.0 / _H)
    h1 = jnp.where(hm > 0, hm, jnp.exp(jnp.minimum(hm, 0.0)) - 1.0)
    z = jnp.dot(h1, w_ref[...], preferred_element_type=jnp.float32)
    z_ref[...] = z
    zh = z.reshape(_NROW_BLK, _H, _NCLS)
    es = jnp.sum(zh * as_ref[...][None], axis=-1)
    ed = jnp.sum(zh * ad_ref[...][None], axis=-1)
    s_ref[...] = jnp.repeat(es, 16, axis=1)
    d_ref[...] = jnp.repeat(ed, 16, axis=1)


def _mid(out1t, sden1, W2, a_src2, a_dst2):
    return pl.pallas_call(
        _mid_body,
        grid=(_N // _NROW_BLK,),
        in_specs=[
            pl.BlockSpec((_H, _NROW_BLK, _HID), lambda i: (0, i, 0)),
            pl.BlockSpec((_NROW_BLK, 128), lambda i: (i, 0)),
            pl.BlockSpec((_HID, _H * _NCLS), lambda i: (0, 0)),
            pl.BlockSpec((_H, _NCLS), lambda i: (0, 0)),
            pl.BlockSpec((_H, _NCLS), lambda i: (0, 0)),
        ],
        out_specs=[
            pl.BlockSpec((_NROW_BLK, _H * _NCLS), lambda i: (i, 0)),
            pl.BlockSpec((_NROW_BLK, 128), lambda i: (i, 0)),
            pl.BlockSpec((_NROW_BLK, 128), lambda i: (i, 0)),
        ],
        out_shape=[
            jax.ShapeDtypeStruct((_N, _H * _NCLS), jnp.float32),
            jax.ShapeDtypeStruct((_N, 128), jnp.float32),
            jax.ShapeDtypeStruct((_N, 128), jnp.float32),
        ],
    )(out1t, sden1, W2, a_src2, a_dst2)


def _final_body(o_ref, sd_ref, out_ref):
    s = sd_ref[...].reshape(_NROW_BLK, _H, 16)[:, :, 0]
    r = 1.0 / (s + 1e-16)
    o = o_ref[...].reshape(_NROW_BLK, _H, _NCLS) * r[:, :, None]
    hm = jnp.mean(o, axis=1)
    m = jnp.max(hm, axis=1, keepdims=True)
    ex = jnp.exp(hm - m)
    out_ref[...] = ex / jnp.sum(ex, axis=1, keepdims=True)


def _final(out2, sden2):
    return pl.pallas_call(
        _final_body,
        grid=(_N // _NROW_BLK,),
        in_specs=[
            pl.BlockSpec((_NROW_BLK, _H * _NCLS), lambda i: (i, 0)),
            pl.BlockSpec((_NROW_BLK, 128), lambda i: (i, 0)),
        ],
        out_specs=pl.BlockSpec((_NROW_BLK, _NCLS), lambda i: (i, 0)),
        out_shape=jax.ShapeDtypeStruct((_N, _NCLS), jnp.float32),
    )(out2, sden2)


_MESH = dict(core_axis_name="c", subcore_axis_name="s")


def _zero_acc(sid, z64, acc, rows):
    for i in range((rows // 64 + 15) // 16):
        start = jnp.minimum((sid + 16 * i) * 64, rows - 64)
        pltpu.sync_copy(z64.at[...], acc.at[pl.ds(start, 64)])


def _copy_out(sid, acc, hbm, rows, rowbase):
    for i in range((rows // 64 + 15) // 16):
        start = jnp.minimum((sid + 16 * i) * 64, rows - 64)
        pltpu.sync_copy(acc.at[pl.ds(start, 64)],
                        hbm.at[pl.ds(rowbase + start, 64)])


def _edgeA_body(ei, SD, ex_hbm, sden_hbm,
                scan_s, scan_d, idx2d, sdx, z64, sem, ex_acc):
    sid = lax.axis_index("s")
    zf = jnp.zeros((16,), jnp.float32)

    for j in range(64):
        for k in range(8):
            z64[j, pl.ds(k * 16, 16)] = zf
    _zero_acc(sid, z64, ex_acc, _AROWS)
    plsc.subcore_barrier()

    def _strip(st, _):
        e0 = sid * _EPAD + st * _STRIP
        pltpu.sync_copy(ei.at[pl.ds(e0, _STRIP)],
                        scan_s.at[pl.ds(0, _STRIP)])
        pltpu.sync_copy(ei.at[pl.ds(_ETOT + e0, _STRIP)], scan_d.at[...])

        def _batch(b, _):
            o = pl.multiple_of(b * _B, 16)
            for j in range(_B // 16):
                sl = pl.ds(j * 16, 16)
                s16 = scan_s[pl.ds(pl.multiple_of(b * _B + j * 16, 16), 16)]
                d16 = scan_d[pl.ds(pl.multiple_of(b * _B + j * 16, 16), 16)]
                dc = jnp.minimum(d16, _TRASH)
                idx2d[0, sl] = dc
                idx2d[1, sl] = s16
                idx2d[2, sl] = dc + _DOFF
            pltpu.async_copy(SD.at[idx2d.at[1]], sdx.at[pl.ds(0, _B)],
                             sem).wait()
            pltpu.async_copy(SD.at[idx2d.at[2]], sdx.at[pl.ds(_B, _B)],
                             sem).wait()
            for j in range(_B):
                for k in range(8):
                    sl = pl.ds(k * 16, 16)
                    t = sdx[j, sl] + sdx[_B + j, sl]
                    t = jnp.maximum(t, 0.2 * t)
                    sdx[2 * _B + j, sl] = jnp.exp(t)
            eo = e0 + b * _B
            pltpu.sync_copy(sdx.at[pl.ds(2 * _B, _B)],
                            ex_hbm.at[pl.ds(eo, _B)])
            pltpu.sync_copy(sdx.at[pl.ds(2 * _B, _B)],
                            ex_acc.at[idx2d.at[0]], add=True)
            return 0

        lax.fori_loop(0, _NBATCH, _batch, 0)
        return 0

    lax.fori_loop(0, _NSTRIP, _strip, 0)
    plsc.subcore_barrier()
    _copy_out(sid, ex_acc, sden_hbm, _AROWS, 0)


def _edgeB_body(ei, exa, Zf, out_hbm,
                scan_s, scan_d, idx2d, zbuf, exbuf, z64, sem, out_acc):
    cid = lax.axis_index("c")
    sid = lax.axis_index("s")
    zf = jnp.zeros((16,), jnp.float32)

    for j in range(64):
        for k in range(8):
            z64[j, pl.ds(k * 16, 16)] = zf
    pltpu.sync_copy(ei.at[pl.ds(2 * _ETOT, 256)],
                    scan_s.at[pl.ds(_BSTASH, 256)])

    for p in range(4):
        _zero_acc(sid, z64, out_acc, _AROWS)
        plsc.subcore_barrier()
        hsel = cid * 4 + p

        def _strip(st, _):
            e0 = sid * _EPAD + st * _STRIP
            pltpu.sync_copy(ei.at[pl.ds(e0, _STRIP)],
                            scan_s.at[pl.ds(0, _STRIP)])
            pltpu.sync_copy(ei.at[pl.ds(_ETOT + e0, _STRIP)],
                            scan_d.at[...])

            def _batch(b, _):
                hoff = scan_s[pl.ds(
                    pl.multiple_of(_BSTASH + hsel * 16, 16), 16)]
                for j in range(_B // 16):
                    sl = pl.ds(j * 16, 16)
                    s16 = scan_s[pl.ds(
                        pl.multiple_of(b * _B + j * 16, 16), 16)]
                    d16 = scan_d[pl.ds(
                        pl.multiple_of(b * _B + j * 16, 16), 16)]
                    idx2d[0, sl] = jnp.minimum(d16, _TRASH)
                    idx2d[1, sl] = s16 + hoff
                zcp = pltpu.async_copy(Zf.at[idx2d.at[1]], zbuf.at[...],
                                       sem)
                eo = e0 + b * _B
                pltpu.sync_copy(exa.at[pl.ds(eo, _B)], exbuf.at[...])
                zcp.wait()
                hh = pl.multiple_of(hsel * 16, 16)
                for j in range(_B):
                    w = exbuf[j, pl.ds(hh, 16)]
                    for k in range(8):
                        sl = pl.ds(k * 16, 16)
                        zbuf[j, sl] = zbuf[j, sl] * w
                pltpu.sync_copy(zbuf.at[...], out_acc.at[idx2d.at[0]],
                                add=True)
                return 0

            lax.fori_loop(0, _NBATCH, _batch, 0)
            return 0

        lax.fori_loop(0, _NSTRIP, _strip, 0)
        plsc.subcore_barrier()
        _copy_out(sid, out_acc, out_hbm, _AROWS, hsel * _AROWS)
        plsc.subcore_barrier()


_C2 = 5120
_A2 = 5184


def _edgeC_body(ei, SD, Z2, out_hbm, sden_hbm,
                scan_s, scan_d, idx2d, sdx, zbuf, z64, sem,
                out_acc, sden_acc):
    cid = lax.axis_index("c")
    sid = lax.axis_index("s")
    zf = jnp.zeros((16,), jnp.float32)

    for j in range(64):
        for k in range(8):
            z64[j, pl.ds(k * 16, 16)] = zf
    pltpu.sync_copy(ei.at[pl.ds(2 * _ETOT, 256)],
                    scan_s.at[pl.ds(_BSTASH, 256)])
    _zero_acc(sid, z64, out_acc, _A2)
    _zero_acc(sid, z64, sden_acc, _A2)
    plsc.subcore_barrier()
    base = cid * _C2

    def _strip(st, _):
        e0 = sid * _EPAD + st * _STRIP
        pltpu.sync_copy(ei.at[pl.ds(e0, _STRIP)],
                        scan_s.at[pl.ds(0, _STRIP)])
        pltpu.sync_copy(ei.at[pl.ds(_ETOT + e0, _STRIP)], scan_d.at[...])

        def _batch(b, _):
            bvec = scan_s[pl.ds(
                pl.multiple_of(_BSTASH + cid * 16, 16), 16)]
            for j in range(_B // 16):
                sl = pl.ds(j * 16, 16)
                s16 = scan_s[pl.ds(
                    pl.multiple_of(b * _B + j * 16, 16), 16)]
                d16 = scan_d[pl.ds(
                    pl.multiple_of(b * _B + j * 16, 16), 16)]
                m = (d16 >= bvec) & (d16 < bvec + _C2)
                dc = jnp.minimum(d16, _TRASH)
                idx2d[0, sl] = jnp.where(m, d16 - bvec, _C2)
                idx2d[1, sl] = s16
                idx2d[2, sl] = dc + _DOFF
            pltpu.async_copy(SD.at[idx2d.at[1]], sdx.at[pl.ds(0, _B)],
                             sem).wait()
            pltpu.async_copy(SD.at[idx2d.at[2]], sdx.at[pl.ds(_B, _B)],
                             sem).wait()
            zcp = pltpu.async_copy(Z2.at[idx2d.at[1]], zbuf.at[...], sem)
            for j in range(_B):
                for k in range(8):
                    sl = pl.ds(k * 16, 16)
                    t = sdx[j, sl] + sdx[_B + j, sl]
                    t = jnp.maximum(t, 0.2 * t)
                    sdx[2 * _B + j, sl] = jnp.exp(t)
            zcp.wait()
            for j in range(_B):
                for k in range(8):
                    sl = pl.ds(k * 16, 16)
                    zbuf[j, sl] = zbuf[j, sl] * sdx[2 * _B + j, sl]
            pltpu.sync_copy(zbuf.at[...], out_acc.at[idx2d.at[0]],
                            add=True)
            pltpu.sync_copy(sdx.at[pl.ds(2 * _B, _B)],
                            sden_acc.at[idx2d.at[0]], add=True)
            return 0

        lax.fori_loop(0, _NBATCH, _batch, 0)
        return 0

    lax.fori_loop(0, _NSTRIP, _strip, 0)
    plsc.subcore_barrier()
    _copy_out(sid, out_acc, out_hbm, _A2, cid * _A2)
    _copy_out(sid, sden_acc, sden_hbm, _A2, cid * _A2)


def _make_kernelA():
    return pl.kernel(
        _edgeA_body,
        out_type=[
            jax.ShapeDtypeStruct((_ETOT, 128), jnp.float32),
            jax.ShapeDtypeStruct((_AROWS, 128), jnp.float32),
        ],
        mesh=plsc.VectorSubcoreMesh(**_MESH),
        scratch_types=[
            pltpu.VMEM((_STRIP,), jnp.int32),
            pltpu.VMEM((_STRIP,), jnp.int32),
            pltpu.VMEM((3, _B), jnp.int32),
            pltpu.VMEM((3 * _B, 128), jnp.float32),
            pltpu.VMEM((64, 128), jnp.float32),
            pltpu.SemaphoreType.DMA,
            pltpu.VMEM_SHARED((_AROWS, 128), jnp.float32),
        ],
    )


def _make_kernelB():
    return pl.kernel(
        _edgeB_body,
        out_type=jax.ShapeDtypeStruct((_H * _AROWS, 128), jnp.float32),
        mesh=plsc.VectorSubcoreMesh(**_MESH),
        scratch_types=[
            pltpu.VMEM((_BSTASH + 256,), jnp.int32),
            pltpu.VMEM((_STRIP,), jnp.int32),
            pltpu.VMEM((2, _B), jnp.int32),
            pltpu.VMEM((_B, 128), jnp.float32),
            pltpu.VMEM((_B, 128), jnp.float32),
            pltpu.VMEM((64, 128), jnp.float32),
            pltpu.SemaphoreType.DMA,
            pltpu.VMEM_SHARED((_AROWS, 128), jnp.float32),
        ],
    )


def _make_kernelC():
    return pl.kernel(
        _edgeC_body,
        out_type=[
            jax.ShapeDtypeStruct((2 * _A2, 128), jnp.float32),
            jax.ShapeDtypeStruct((2 * _A2, 128), jnp.float32),
        ],
        mesh=plsc.VectorSubcoreMesh(**_MESH),
        scratch_types=[
            pltpu.VMEM((_BSTASH + 256,), jnp.int32),
            pltpu.VMEM((_STRIP,), jnp.int32),
            pltpu.VMEM((3, _B), jnp.int32),
            pltpu.VMEM((3 * _B, 128), jnp.float32),
            pltpu.VMEM((_B, 128), jnp.float32),
            pltpu.VMEM((64, 128), jnp.float32),
            pltpu.SemaphoreType.DMA,
            pltpu.VMEM_SHARED((_A2, 128), jnp.float32),
            pltpu.VMEM_SHARED((_A2, 128), jnp.float32),
        ],
    )


def _sd_table(S, D):
    return jnp.concatenate([
        S, jnp.zeros((_DOFF - _N, 128), jnp.float32),
        D, jnp.zeros((_SDROWS - _DOFF - _N, 128), jnp.float32)])


def kernel(x, edge_index, W1, a_src1, a_dst1, W2, a_src2, a_dst2):
    epad = _ETOT - _E
    srcs_p = jnp.concatenate([edge_index[0], jnp.zeros(epad, jnp.int32)])
    dsts_p = jnp.concatenate(
        [edge_index[1], jnp.full((epad,), 1 << 20, jnp.int32)])
    tailB = jnp.repeat(jnp.arange(16, dtype=jnp.int32) * _N, 16)
    tailC = jnp.repeat(jnp.arange(16, dtype=jnp.int32) * _C2, 16)
    eiB = jnp.concatenate([srcs_p, dsts_p, tailB])
    eiC = jnp.concatenate([srcs_p, dsts_p, tailC])

    z1t, S1, D1 = _proj1(x, W1, a_src1, a_dst1)
    SD1 = _sd_table(S1, D1)
    exa, sden1 = _make_kernelA()(eiB, SD1)
    out1t = _make_kernelB()(eiB, exa, z1t.reshape(_H * _N, _HID))
    out1t = out1t.reshape(_H, _AROWS, 128)[:, :_N]

    z2, S2, D2 = _mid(out1t, sden1[:_N], W2, a_src2, a_dst2)
    SD2 = _sd_table(S2, D2)
    out2, sden2 = _make_kernelC()(eiC, SD2, z2)
    out2n = jnp.concatenate([out2[:_C2], out2[_A2:_A2 + _N - _C2]])
    sden2n = jnp.concatenate([sden2[:_C2], sden2[_A2:_A2 + _N - _C2]])

    return _final(out2n, sden2n)


# speedup vs baseline: 8.4811x; 1.1000x over previous
import functools

import jax
import jax.numpy as jnp
from jax import lax
from jax.experimental import pallas as pl
from jax.experimental.pallas import tpu as pltpu
from jax.experimental.pallas import tpu_sc as plsc

_N = 10000
_E = 320000
_DIN = 128
_HID = 128
_NCLS = 16
_H = 8

_NROWBLK = 1000
_DOFF = 10240
_SDROWS = 20544
_TRASH = 10240
_AROWS = 10304

_STRIP = 2048
_EPAD = 20480
_ETOT = 16 * _EPAD
_NSTRIP = _EPAD // _STRIP
BA = 64
BC = 32
BSTASH = 2176


def _proj1_body(x_ref, w_ref, as_ref, ad_ref, zt_ref, s_ref, d_ref):
    z = jnp.dot(x_ref[...], w_ref[...], preferred_element_type=jnp.float32)
    zh = z.reshape(_NROWBLK, _H, _HID)
    zt_ref[...] = zh.transpose(1, 0, 2)
    es = jnp.sum(zh * as_ref[...][None], axis=-1)
    ed = jnp.sum(zh * ad_ref[...][None], axis=-1)
    s_ref[...] = jnp.repeat(es, 16, axis=1)
    d_ref[...] = jnp.repeat(ed, 16, axis=1)


def _proj1(x, W1, a_src1, a_dst1):
    return pl.pallas_call(
        _proj1_body,
        grid=(_N // _NROWBLK,),
        in_specs=[
            pl.BlockSpec((_NROWBLK, _DIN), lambda i: (i, 0)),
            pl.BlockSpec((_DIN, _H * _HID), lambda i: (0, 0)),
            pl.BlockSpec((_H, _HID), lambda i: (0, 0)),
            pl.BlockSpec((_H, _HID), lambda i: (0, 0)),
        ],
        out_specs=[
            pl.BlockSpec((_H, _NROWBLK, _HID), lambda i: (0, i, 0)),
            pl.BlockSpec((_NROWBLK, 128), lambda i: (i, 0)),
            pl.BlockSpec((_NROWBLK, 128), lambda i: (i, 0)),
        ],
        out_shape=[
            jax.ShapeDtypeStruct((_H, _N, _HID), jnp.float32),
            jax.ShapeDtypeStruct((_N, 128), jnp.float32),
            jax.ShapeDtypeStruct((_N, 128), jnp.float32),
        ],
    )(x, W1, a_src1, a_dst1)


def _mid_body(o_ref, sd_ref, w_ref, as_ref, ad_ref, z_ref, s_ref, d_ref):
    s = sd_ref[...].reshape(_NROWBLK, _H, 16)[:, :, 0]
    r = 1.0 / (s + 1e-16)
    o = o_ref[...]
    hm = jnp.zeros((_NROWBLK, _HID), jnp.float32)
    for h in range(_H):
        hm = hm + o[h] * r[:, h:h + 1]
    hm = hm * (1.0 / _H)
    h1 = jnp.where(hm > 0, hm, jnp.exp(jnp.minimum(hm, 0.0)) - 1.0)
    z = jnp.dot(h1, w_ref[...], preferred_element_type=jnp.float32)
    z_ref[...] = z
    zh = z.reshape(_NROWBLK, _H, _NCLS)
    es = jnp.sum(zh * as_ref[...][None], axis=-1)
    ed = jnp.sum(zh * ad_ref[...][None], axis=-1)
    s_ref[...] = jnp.repeat(es, 16, axis=1)
    d_ref[...] = jnp.repeat(ed, 16, axis=1)


def _mid(out1t, sden1, W2, a_src2, a_dst2):
    return pl.pallas_call(
        _mid_body,
        grid=(_N // _NROWBLK,),
        in_specs=[
            pl.BlockSpec((_H, _NROWBLK, _HID), lambda i: (0, i, 0)),
            pl.BlockSpec((_NROWBLK, 128), lambda i: (i, 0)),
            pl.BlockSpec((_HID, _H * _NCLS), lambda i: (0, 0)),
            pl.BlockSpec((_H, _NCLS), lambda i: (0, 0)),
            pl.BlockSpec((_H, _NCLS), lambda i: (0, 0)),
        ],
        out_specs=[
            pl.BlockSpec((_NROWBLK, _H * _NCLS), lambda i: (i, 0)),
            pl.BlockSpec((_NROWBLK, 128), lambda i: (i, 0)),
            pl.BlockSpec((_NROWBLK, 128), lambda i: (i, 0)),
        ],
        out_shape=[
            jax.ShapeDtypeStruct((_N, _H * _NCLS), jnp.float32),
            jax.ShapeDtypeStruct((_N, 128), jnp.float32),
            jax.ShapeDtypeStruct((_N, 128), jnp.float32),
        ],
    )(out1t, sden1, W2, a_src2, a_dst2)


def _final_body(o_ref, sd_ref, out_ref):
    s = sd_ref[...].reshape(_NROWBLK, _H, 16)[:, :, 0]
    r = 1.0 / (s + 1e-16)
    o = o_ref[...].reshape(_NROWBLK, _H, _NCLS) * r[:, :, None]
    hm = jnp.mean(o, axis=1)
    m = jnp.max(hm, axis=1, keepdims=True)
    ex = jnp.exp(hm - m)
    out_ref[...] = ex / jnp.sum(ex, axis=1, keepdims=True)


def _final(out2, sden2):
    return pl.pallas_call(
        _final_body,
        grid=(_N // _NROWBLK,),
        in_specs=[
            pl.BlockSpec((_NROWBLK, _H * _NCLS), lambda i: (i, 0)),
            pl.BlockSpec((_NROWBLK, 128), lambda i: (i, 0)),
        ],
        out_specs=pl.BlockSpec((_NROWBLK, _NCLS), lambda i: (i, 0)),
        out_shape=jax.ShapeDtypeStruct((_N, _NCLS), jnp.float32),
    )(out2, sden2)


_MESH = dict(core_axis_name="c", subcore_axis_name="s")


def _zero_acc(sid, z64, acc, rows):
    for i in range((rows // 64 + 15) // 16):
        start = jnp.minimum((sid + 16 * i) * 64, rows - 64)
        pltpu.sync_copy(z64.at[...], acc.at[pl.ds(start, 64)])


def _copy_out(sid, acc, hbm, rows, rowbase):
    for i in range((rows // 64 + 15) // 16):
        start = jnp.minimum((sid + 16 * i) * 64, rows - 64)
        pltpu.sync_copy(acc.at[pl.ds(start, 64)],
                        hbm.at[pl.ds(rowbase + start, 64)])


def _edgeA_body(B, ei, SD, ex_hbm, sden_hbm,
                scan_s, scan_d, idx2d, sdx, z64, sem, ex_acc):
    sid = lax.axis_index("s")
    zf = jnp.zeros((16,), jnp.float32)

    for j in range(64):
        for k in range(8):
            z64[j, pl.ds(k * 16, 16)] = zf
    _zero_acc(sid, z64, ex_acc, _AROWS)
    plsc.subcore_barrier()

    def _strip(st, _):
        e0 = sid * _EPAD + st * _STRIP
        pltpu.sync_copy(ei.at[pl.ds(e0, _STRIP)],
                        scan_s.at[pl.ds(0, _STRIP)])
        pltpu.sync_copy(ei.at[pl.ds(_ETOT + e0, _STRIP)], scan_d.at[...])

        def _batch(b, _):
            o = pl.multiple_of(b * B, 16)
            for j in range(B // 16):
                sl = pl.ds(j * 16, 16)
                s16 = scan_s[pl.ds(pl.multiple_of(b * B + j * 16, 16), 16)]
                d16 = scan_d[pl.ds(pl.multiple_of(b * B + j * 16, 16), 16)]
                dc = jnp.minimum(d16, _TRASH)
                idx2d[0, sl] = dc
                idx2d[1, sl] = s16
                idx2d[2, sl] = dc + _DOFF
            pltpu.async_copy(SD.at[idx2d.at[1]], sdx.at[pl.ds(0, B)],
                             sem).wait()
            pltpu.async_copy(SD.at[idx2d.at[2]], sdx.at[pl.ds(B, B)],
                             sem).wait()
            for j in range(B):
                for k in range(8):
                    sl = pl.ds(k * 16, 16)
                    t = sdx[j, sl] + sdx[B + j, sl]
                    t = jnp.maximum(t, 0.2 * t)
                    sdx[2 * B + j, sl] = jnp.exp(t)
            eo = e0 + b * B
            pltpu.sync_copy(sdx.at[pl.ds(2 * B, B)],
                            ex_hbm.at[pl.ds(eo, B)])
            pltpu.sync_copy(sdx.at[pl.ds(2 * B, B)],
                            ex_acc.at[idx2d.at[0]], add=True)
            return 0

        lax.fori_loop(0, _STRIP // B, _batch, 0)
        return 0

    lax.fori_loop(0, _NSTRIP, _strip, 0)
    plsc.subcore_barrier()
    _copy_out(sid, ex_acc, sden_hbm, _AROWS, 0)


def _edgeB_body(B, ei, exa, Zf, out_hbm,
                scan_s, scan_d, idx2d, zbuf, exbuf, z64, sem, out_acc):
    cid = lax.axis_index("c")
    sid = lax.axis_index("s")
    zf = jnp.zeros((16,), jnp.float32)

    for j in range(64):
        for k in range(8):
            z64[j, pl.ds(k * 16, 16)] = zf
    pltpu.sync_copy(ei.at[pl.ds(2 * _ETOT, 256)],
                    scan_s.at[pl.ds(BSTASH, 256)])

    for p in range(4):
        _zero_acc(sid, z64, out_acc, _AROWS)
        plsc.subcore_barrier()
        hsel = cid * 4 + p

        def _strip(st, _):
            e0 = sid * _EPAD + st * _STRIP
            pltpu.sync_copy(ei.at[pl.ds(e0, _STRIP)],
                            scan_s.at[pl.ds(0, _STRIP)])
            pltpu.sync_copy(ei.at[pl.ds(_ETOT + e0, _STRIP)],
                            scan_d.at[...])

            def _batch(b, _):
                hoff = scan_s[pl.ds(
                    pl.multiple_of(BSTASH + hsel * 16, 16), 16)]
                for j in range(B // 16):
                    sl = pl.ds(j * 16, 16)
                    s16 = scan_s[pl.ds(
                        pl.multiple_of(b * B + j * 16, 16), 16)]
                    d16 = scan_d[pl.ds(
                        pl.multiple_of(b * B + j * 16, 16), 16)]
                    idx2d[0, sl] = jnp.minimum(d16, _TRASH)
                    idx2d[1, sl] = s16 + hoff
                zcp = pltpu.async_copy(Zf.at[idx2d.at[1]], zbuf.at[...],
                                       sem)
                eo = e0 + b * B
                pltpu.sync_copy(exa.at[pl.ds(eo, B)], exbuf.at[...])
                zcp.wait()
                hh = pl.multiple_of(hsel * 16, 16)
                for j in range(B):
                    w = exbuf[j, pl.ds(hh, 16)]
                    for k in range(8):
                        sl = pl.ds(k * 16, 16)
                        zbuf[j, sl] = zbuf[j, sl] * w
                pltpu.sync_copy(zbuf.at[...], out_acc.at[idx2d.at[0]],
                                add=True)
                return 0

            lax.fori_loop(0, _STRIP // B, _batch, 0)
            return 0

        lax.fori_loop(0, _NSTRIP, _strip, 0)
        plsc.subcore_barrier()
        _copy_out(sid, out_acc, out_hbm, _AROWS, hsel * _AROWS)
        plsc.subcore_barrier()


_C2 = 5120
_A2 = 5184


def _edgeC_body(B, ei, SD, Z2, out_hbm, sden_hbm,
                scan_s, scan_d, idx2d, sdx, zbuf, z64, sem,
                out_acc, sden_acc):
    cid = lax.axis_index("c")
    sid = lax.axis_index("s")
    zf = jnp.zeros((16,), jnp.float32)

    for j in range(64):
        for k in range(8):
            z64[j, pl.ds(k * 16, 16)] = zf
    pltpu.sync_copy(ei.at[pl.ds(2 * _ETOT, 256)],
                    scan_s.at[pl.ds(BSTASH, 256)])
    _zero_acc(sid, z64, out_acc, _A2)
    _zero_acc(sid, z64, sden_acc, _A2)
    plsc.subcore_barrier()
    base = cid * _C2

    def _strip(st, _):
        e0 = sid * _EPAD + st * _STRIP
        pltpu.sync_copy(ei.at[pl.ds(e0, _STRIP)],
                        scan_s.at[pl.ds(0, _STRIP)])
        pltpu.sync_copy(ei.at[pl.ds(_ETOT + e0, _STRIP)], scan_d.at[...])

        def _batch(b, _):
            bvec = scan_s[pl.ds(
                pl.multiple_of(BSTASH + cid * 16, 16), 16)]
            for j in range(B // 16):
                sl = pl.ds(j * 16, 16)
                s16 = scan_s[pl.ds(
                    pl.multiple_of(b * B + j * 16, 16), 16)]
                d16 = scan_d[pl.ds(
                    pl.multiple_of(b * B + j * 16, 16), 16)]
                m = (d16 >= bvec) & (d16 < bvec + _C2)
                dc = jnp.minimum(d16, _TRASH)
                idx2d[0, sl] = jnp.where(m, d16 - bvec, _C2)
                idx2d[1, sl] = s16
                idx2d[2, sl] = dc + _DOFF
            pltpu.async_copy(SD.at[idx2d.at[1]], sdx.at[pl.ds(0, B)],
                             sem).wait()
            pltpu.async_copy(SD.at[idx2d.at[2]], sdx.at[pl.ds(B, B)],
                             sem).wait()
            zcp = pltpu.async_copy(Z2.at[idx2d.at[1]], zbuf.at[...], sem)
            for j in range(B):
                for k in range(8):
                    sl = pl.ds(k * 16, 16)
                    t = sdx[j, sl] + sdx[B + j, sl]
                    t = jnp.maximum(t, 0.2 * t)
                    sdx[2 * B + j, sl] = jnp.exp(t)
            zcp.wait()
            for j in range(B):
                for k in range(8):
                    sl = pl.ds(k * 16, 16)
                    zbuf[j, sl] = zbuf[j, sl] * sdx[2 * B + j, sl]
            pltpu.sync_copy(zbuf.at[...], out_acc.at[idx2d.at[0]],
                            add=True)
            pltpu.sync_copy(sdx.at[pl.ds(2 * B, B)],
                            sden_acc.at[idx2d.at[0]], add=True)
            return 0

        lax.fori_loop(0, _STRIP // B, _batch, 0)
        return 0

    lax.fori_loop(0, _NSTRIP, _strip, 0)
    plsc.subcore_barrier()
    _copy_out(sid, out_acc, out_hbm, _A2, cid * _A2)
    _copy_out(sid, sden_acc, sden_hbm, _A2, cid * _A2)


def _make_kernelA():
    B = BA
    return pl.kernel(
        functools.partial(_edgeA_body, B),
        out_type=[
            jax.ShapeDtypeStruct((_ETOT, 128), jnp.float32),
            jax.ShapeDtypeStruct((_AROWS, 128), jnp.float32),
        ],
        mesh=plsc.VectorSubcoreMesh(**_MESH),
        scratch_types=[
            pltpu.VMEM((_STRIP,), jnp.int32),
            pltpu.VMEM((_STRIP,), jnp.int32),
            pltpu.VMEM((3, B), jnp.int32),
            pltpu.VMEM((3 * B, 128), jnp.float32),
            pltpu.VMEM((64, 128), jnp.float32),
            pltpu.SemaphoreType.DMA,
            pltpu.VMEM_SHARED((_AROWS, 128), jnp.float32),
        ],
    )


def _make_kernelB():
    B = BA
    return pl.kernel(
        functools.partial(_edgeB_body, B),
        out_type=jax.ShapeDtypeStruct((_H * _AROWS, 128), jnp.float32),
        mesh=plsc.VectorSubcoreMesh(**_MESH),
        scratch_types=[
            pltpu.VMEM((BSTASH + 256,), jnp.int32),
            pltpu.VMEM((_STRIP,), jnp.int32),
            pltpu.VMEM((2, B), jnp.int32),
            pltpu.VMEM((B, 128), jnp.float32),
            pltpu.VMEM((B, 128), jnp.float32),
            pltpu.VMEM((64, 128), jnp.float32),
            pltpu.SemaphoreType.DMA,
            pltpu.VMEM_SHARED((_AROWS, 128), jnp.float32),
        ],
    )


def _make_kernelC():
    B = BC
    return pl.kernel(
        functools.partial(_edgeC_body, B),
        out_type=[
            jax.ShapeDtypeStruct((2 * _A2, 128), jnp.float32),
            jax.ShapeDtypeStruct((2 * _A2, 128), jnp.float32),
        ],
        mesh=plsc.VectorSubcoreMesh(**_MESH),
        scratch_types=[
            pltpu.VMEM((BSTASH + 256,), jnp.int32),
            pltpu.VMEM((_STRIP,), jnp.int32),
            pltpu.VMEM((3, B), jnp.int32),
            pltpu.VMEM((3 * B, 128), jnp.float32),
            pltpu.VMEM((B, 128), jnp.float32),
            pltpu.VMEM((64, 128), jnp.float32),
            pltpu.SemaphoreType.DMA,
            pltpu.VMEM_SHARED((_A2, 128), jnp.float32),
            pltpu.VMEM_SHARED((_A2, 128), jnp.float32),
        ],
    )


def _sd_table(S, D):
    return jnp.concatenate([
        S, jnp.zeros((_DOFF - _N, 128), jnp.float32),
        D, jnp.zeros((_SDROWS - _DOFF - _N, 128), jnp.float32)])


def kernel(x, edge_index, W1, a_src1, a_dst1, W2, a_src2, a_dst2):
    epad = _ETOT - _E
    srcs_p = jnp.concatenate([edge_index[0], jnp.zeros(epad, jnp.int32)])
    dsts_p = jnp.concatenate(
        [edge_index[1], jnp.full((epad,), 1 << 20, jnp.int32)])
    tailB = jnp.repeat(jnp.arange(16, dtype=jnp.int32) * _N, 16)
    tailC = jnp.repeat(jnp.arange(16, dtype=jnp.int32) * _C2, 16)
    eiB = jnp.concatenate([srcs_p, dsts_p, tailB])
    eiC = jnp.concatenate([srcs_p, dsts_p, tailC])

    z1t, S1, D1 = _proj1(x, W1, a_src1, a_dst1)
    SD1 = _sd_table(S1, D1)
    exa, sden1 = _make_kernelA()(eiB, SD1)
    out1t = _make_kernelB()(eiB, exa, z1t.reshape(_H * _N, _HID))
    out1t = out1t.reshape(_H, _AROWS, 128)[:, :_N]

    z2, S2, D2 = _mid(out1t, sden1[:_N], W2, a_src2, a_dst2)
    SD2 = _sd_table(S2, D2)
    out2, sden2 = _make_kernelC()(eiC, SD2, z2)
    out2n = jnp.concatenate([out2[:_C2], out2[_A2:_A2 + _N - _C2]])
    sden2n = jnp.concatenate([sden2[:_C2], sden2[_A2:_A2 + _N - _C2]])

    return _final(out2n, sden2n)
